# baseline (speedup 1.0000x reference)
import jax
import jax.numpy as jnp
from jax.experimental import pallas as pl
from jax.experimental.pallas import tpu as pltpu

_NEG_SLOPE = 0.2


def _feat_kernel(x_ref, w_ref, a1_ref, a2_ref,
                 h_ref, u_ref, p_ref, v_ref, q_ref):
    h = jnp.dot(x_ref[...], w_ref[...], preferred_element_type=jnp.float32)
    h_ref[...] = h
    s = jnp.dot(h, a1_ref[...], preferred_element_type=jnp.float32)
    t = jnp.dot(h, a2_ref[...], preferred_element_type=jnp.float32)
    u_ref[...] = jnp.exp(-s)
    p_ref[...] = jnp.exp(-_NEG_SLOPE * s)
    v_ref[...] = jnp.exp(-t)
    q_ref[...] = jnp.exp(-_NEG_SLOPE * t)


def _gat_kernel(adj_ref, h_ref, u_ref, p_ref, vT_ref, qT_ref, out_ref):
    i = pl.program_id(0)
    br = adj_ref.shape[0]

    u = u_ref[...]
    p = p_ref[...]
    vT = vT_ref[...]
    qT = qT_ref[...]

    val = jnp.minimum(u * vT, p * qT)
    pm = adj_ref[...] * val

    acc = jnp.dot(pm, h_ref[...], preferred_element_type=jnp.float32)
    rs = jnp.sum(pm, axis=1, keepdims=True)

    flag = rs == 0.0
    h_i = h_ref[pl.ds(i * br, br), :]
    hp = jnp.where(flag, h_i, acc) / jnp.where(flag, 1.0, rs)
    out_ref[...] = jnp.where(hp > 0.0, hp, jnp.exp(hp) - 1.0)


def _pick_block(n, targets):
    for t in targets:
        if n % t == 0:
            return t
    return n


def kernel(input, adj, W, a):
    n, _ = input.shape
    fo = W.shape[1]
    a1 = a[:, :fo].reshape(fo, 1)
    a2 = a[:, fo:].reshape(fo, 1)

    bh = _pick_block(n, (2000, 1000, 200, 8))
    h, u, p, v, q = pl.pallas_call(
        _feat_kernel,
        grid=(n // bh,),
        in_specs=[
            pl.BlockSpec((bh, input.shape[1]), lambda i: (i, 0)),
            pl.BlockSpec((W.shape[0], fo), lambda i: (0, 0)),
            pl.BlockSpec((fo, 1), lambda i: (0, 0)),
            pl.BlockSpec((fo, 1), lambda i: (0, 0)),
        ],
        out_specs=[
            pl.BlockSpec((bh, fo), lambda i: (i, 0)),
            pl.BlockSpec((bh, 1), lambda i: (i, 0)),
            pl.BlockSpec((bh, 1), lambda i: (i, 0)),
            pl.BlockSpec((bh, 1), lambda i: (i, 0)),
            pl.BlockSpec((bh, 1), lambda i: (i, 0)),
        ],
        out_shape=[
            jax.ShapeDtypeStruct((n, fo), jnp.float32),
            jax.ShapeDtypeStruct((n, 1), jnp.float32),
            jax.ShapeDtypeStruct((n, 1), jnp.float32),
            jax.ShapeDtypeStruct((n, 1), jnp.float32),
            jax.ShapeDtypeStruct((n, 1), jnp.float32),
        ],
    )(input, W, a1, a2)

    vT = v.reshape(1, n)
    qT = q.reshape(1, n)

    br = _pick_block(n, (200, 80, 8))
    out = pl.pallas_call(
        _gat_kernel,
        grid=(n // br,),
        in_specs=[
            pl.BlockSpec((br, n), lambda i: (i, 0)),
            pl.BlockSpec((n, fo), lambda i: (0, 0)),
            pl.BlockSpec((br, 1), lambda i: (i, 0)),
            pl.BlockSpec((br, 1), lambda i: (i, 0)),
            pl.BlockSpec((1, n), lambda i: (0, 0)),
            pl.BlockSpec((1, n), lambda i: (0, 0)),
        ],
        out_specs=pl.BlockSpec((br, fo), lambda i: (i, 0)),
        out_shape=jax.ShapeDtypeStruct((n, fo), jnp.float32),
        compiler_params=pltpu.CompilerParams(
            dimension_semantics=("arbitrary",),
        ),
    )(adj, h, u, p, vT, qT)
    return out


# speedup vs baseline: 1.1806x; 1.1806x over previous
import jax
import jax.numpy as jnp
from jax.experimental import pallas as pl
from jax.experimental.pallas import tpu as pltpu

_NEG_SLOPE = 0.2


def _feat_kernel(x_ref, w_ref, a1_ref, a2_ref,
                 h_ref, u_ref, p_ref, v_ref, q_ref):
    fo = w_ref.shape[1]
    h = jnp.dot(x_ref[...], w_ref[...], preferred_element_type=jnp.float32)
    h_ref[:, :fo] = h
    h_ref[:, fo:] = jnp.ones((h.shape[0], 1), jnp.float32)
    s = jnp.dot(h, a1_ref[...], preferred_element_type=jnp.float32)
    t = jnp.dot(h, a2_ref[...], preferred_element_type=jnp.float32)
    u_ref[...] = jnp.exp(-s)
    p_ref[...] = jnp.exp(-_NEG_SLOPE * s)
    v_ref[...] = jnp.exp(-t)
    q_ref[...] = jnp.exp(-_NEG_SLOPE * t)


def _gat_kernel(adj_ref, h_ref, u_ref, p_ref, vT_ref, qT_ref, out_ref):
    i = pl.program_id(0)
    br = adj_ref.shape[0]
    fo = h_ref.shape[1] - 1

    u = u_ref[...]
    p = p_ref[...]
    vT = vT_ref[...]
    qT = qT_ref[...]

    val = jnp.minimum(u * vT, p * qT)
    pm = adj_ref[...] * val

    accs = jnp.dot(pm, h_ref[...], preferred_element_type=jnp.float32)
    acc = accs[:, :fo]
    rs = accs[:, fo:]

    flag = rs == 0.0
    h_i = h_ref[pl.ds(i * br, br), :fo]
    hp = jnp.where(flag, h_i, acc) / jnp.where(flag, 1.0, rs)
    out_ref[...] = jnp.where(hp > 0.0, hp, jnp.exp(hp) - 1.0)


def _pick_block(n, targets):
    for t in targets:
        if n % t == 0:
            return t
    return n


def kernel(input, adj, W, a):
    n, _ = input.shape
    fo = W.shape[1]
    a1 = a[:, :fo].reshape(fo, 1)
    a2 = a[:, fo:].reshape(fo, 1)

    bh = _pick_block(n, (2000, 1000, 200, 8))
    h, u, p, v, q = pl.pallas_call(
        _feat_kernel,
        grid=(n // bh,),
        in_specs=[
            pl.BlockSpec((bh, input.shape[1]), lambda i: (i, 0)),
            pl.BlockSpec((W.shape[0], fo), lambda i: (0, 0)),
            pl.BlockSpec((fo, 1), lambda i: (0, 0)),
            pl.BlockSpec((fo, 1), lambda i: (0, 0)),
        ],
        out_specs=[
            pl.BlockSpec((bh, fo + 1), lambda i: (i, 0)),
            pl.BlockSpec((bh, 1), lambda i: (i, 0)),
            pl.BlockSpec((bh, 1), lambda i: (i, 0)),
            pl.BlockSpec((bh, 1), lambda i: (i, 0)),
            pl.BlockSpec((bh, 1), lambda i: (i, 0)),
        ],
        out_shape=[
            jax.ShapeDtypeStruct((n, fo + 1), jnp.float32),
            jax.ShapeDtypeStruct((n, 1), jnp.float32),
            jax.ShapeDtypeStruct((n, 1), jnp.float32),
            jax.ShapeDtypeStruct((n, 1), jnp.float32),
            jax.ShapeDtypeStruct((n, 1), jnp.float32),
        ],
    )(input, W, a1, a2)

    vT = v.reshape(1, n)
    qT = q.reshape(1, n)

    br = _pick_block(n, (400, 200, 80, 8))
    out = pl.pallas_call(
        _gat_kernel,
        grid=(n // br,),
        in_specs=[
            pl.BlockSpec((br, n), lambda i: (i, 0)),
            pl.BlockSpec((n, fo + 1), lambda i: (0, 0)),
            pl.BlockSpec((br, 1), lambda i: (i, 0)),
            pl.BlockSpec((br, 1), lambda i: (i, 0)),
            pl.BlockSpec((1, n), lambda i: (0, 0)),
            pl.BlockSpec((1, n), lambda i: (0, 0)),
        ],
        out_specs=pl.BlockSpec((br, fo), lambda i: (i, 0)),
        out_shape=jax.ShapeDtypeStruct((n, fo), jnp.float32),
        compiler_params=pltpu.CompilerParams(
            dimension_semantics=("arbitrary",),
        ),
    )(adj, h, u, p, vT, qT)
    return out


# speedup vs baseline: 1.4323x; 1.2132x over previous
import jax
import jax.numpy as jnp
from jax.experimental import pallas as pl
from jax.experimental.pallas import tpu as pltpu

_NEG_SLOPE = 0.2


def _gat_kernel(adj_ref, x_ref, w_ref, a1_ref, a2_ref, out_ref,
                ha_s, u_s, p_s, vT_s, qT_s):
    i = pl.program_id(0)
    br = adj_ref.shape[0]
    fo = w_ref.shape[1]

    @pl.when(i == 0)
    def _():
        h = jnp.dot(x_ref[...], w_ref[...], preferred_element_type=jnp.float32)
        ha_s[:, :fo] = h
        ha_s[:, fo:] = jnp.ones((h.shape[0], 1), jnp.float32)
        s = jnp.dot(h, a1_ref[...], preferred_element_type=jnp.float32)
        t = jnp.dot(h, a2_ref[...], preferred_element_type=jnp.float32)
        u_s[...] = jnp.exp(-s)
        p_s[...] = jnp.exp(-_NEG_SLOPE * s)
        vT_s[...] = jnp.transpose(jnp.exp(-t))
        qT_s[...] = jnp.transpose(jnp.exp(-_NEG_SLOPE * t))

    u = u_s[pl.ds(i * br, br), :]
    p = p_s[pl.ds(i * br, br), :]

    val = jnp.minimum(u * vT_s[...], p * qT_s[...])
    pm = adj_ref[...] * val

    accs = jnp.dot(pm, ha_s[...], preferred_element_type=jnp.float32)
    acc = accs[:, :fo]
    rs = accs[:, fo:]

    flag = rs == 0.0
    h_i = ha_s[pl.ds(i * br, br), :fo]
    hp = jnp.where(flag, h_i, acc) / jnp.where(flag, 1.0, rs)
    out_ref[...] = jnp.where(hp > 0.0, hp, jnp.exp(hp) - 1.0)


def _pick_block(n, targets):
    for t in targets:
        if n % t == 0:
            return t
    return n


def kernel(input, adj, W, a):
    n, _ = input.shape
    fo = W.shape[1]
    a1 = a[:, :fo].reshape(fo, 1)
    a2 = a[:, fo:].reshape(fo, 1)

    br = _pick_block(n, (400, 200, 80, 8))
    out = pl.pallas_call(
        _gat_kernel,
        grid=(n // br,),
        in_specs=[
            pl.BlockSpec((br, n), lambda i: (i, 0)),
            pl.BlockSpec((n, input.shape[1]), lambda i: (0, 0)),
            pl.BlockSpec((W.shape[0], fo), lambda i: (0, 0)),
            pl.BlockSpec((fo, 1), lambda i: (0, 0)),
            pl.BlockSpec((fo, 1), lambda i: (0, 0)),
        ],
        out_specs=pl.BlockSpec((br, fo), lambda i: (i, 0)),
        out_shape=jax.ShapeDtypeStruct((n, fo), jnp.float32),
        scratch_shapes=[
            pltpu.VMEM((n, fo + 1), jnp.float32),
            pltpu.VMEM((n, 1), jnp.float32),
            pltpu.VMEM((n, 1), jnp.float32),
            pltpu.VMEM((1, n), jnp.float32),
            pltpu.VMEM((1, n), jnp.float32),
        ],
        compiler_params=pltpu.CompilerParams(
            dimension_semantics=("arbitrary",),
        ),
    )(adj, input, W, a1, a2)
    return out
